# Initial kernel scaffold; baseline (speedup 1.0000x reference)
import jax
import jax.numpy as jnp
from jax.experimental import pallas as pl


def kernel(per_atom_charge, positions, atomic_subsystem_indices, per_system_energy):
    raise NotImplementedError("write your pallas kernel here")


# speedup vs baseline: 1.5120x; 1.5120x over previous
import functools

import jax
import jax.numpy as jnp
from jax import lax
from jax.experimental import pallas as pl
from jax.experimental.pallas import tpu as pltpu
from jax.experimental.pallas import tpu_sc as plsc

N_ATOMS = 1_600_000
N_SYS = 100_000
NC = 2
NS = 16
NW = NC * NS
APW = N_ATOMS // NW
B = 10_000
NBLK = APW // B
L = 16

STRIPE = 6_240
TAIL_OFF = NS * STRIPE
TAIL = N_SYS - TAIL_OFF
RCHUNK = 3_120


def _sc_partials(charge, pos3, idx, zblock):
  mesh = plsc.VectorSubcoreMesh(core_axis_name="c", subcore_axis_name="s")

  @functools.partial(
      pl.kernel,
      out_type=jax.ShapeDtypeStruct((NC * 4 * N_SYS,), jnp.float32),
      mesh=mesh,
      compiler_params=pltpu.CompilerParams(needs_layout_passes=False),
      scratch_types=[
          pltpu.VMEM_SHARED((N_SYS,), jnp.float32),
          pltpu.VMEM_SHARED((N_SYS,), jnp.float32),
          pltpu.VMEM_SHARED((N_SYS,), jnp.float32),
          pltpu.VMEM_SHARED((N_SYS,), jnp.float32),
          pltpu.VMEM((B,), jnp.float32),
          pltpu.VMEM((B,), jnp.float32),
          pltpu.VMEM((B,), jnp.float32),
          pltpu.VMEM((B,), jnp.float32),
          pltpu.VMEM((3 * B,), jnp.float32),
          pltpu.VMEM((B,), jnp.int32),
      ],
  )
  def k(q_hbm, pos_hbm, idx_hbm, z_hbm, out_hbm,
        acc0, acc1, acc2, acc3, q_v, x_v, y_v, z_v, pos_v, idx_v):
    cid = lax.axis_index("c")
    sid = lax.axis_index("s")
    wid = cid * NS + sid
    accs = (acc0, acc1, acc2, acc3)

    pltpu.sync_copy(z_hbm, q_v.at[pl.ds(0, STRIPE)])
    for acc in accs:
      pltpu.sync_copy(q_v.at[pl.ds(0, STRIPE)],
                      acc.at[pl.ds(sid * STRIPE, STRIPE)])

    @pl.when(sid == 0)
    def _zero_tail():
      for acc in accs:
        pltpu.sync_copy(q_v.at[pl.ds(0, TAIL)],
                        acc.at[pl.ds(TAIL_OFF, TAIL)])

    plsc.subcore_barrier()

    lanes3 = lax.iota(jnp.int32, L) * 3

    for j in range(NBLK):
      base = pl.multiple_of(wid * APW + j * B, 8)
      pltpu.sync_copy(q_hbm.at[pl.ds(base, B)], q_v)
      pltpu.sync_copy(pos_hbm.at[pl.ds(base * 3, 3 * B)], pos_v)
      pltpu.sync_copy(idx_hbm.at[pl.ds(base, B)], idx_v)

      def step(i, carry):
        r0 = pl.multiple_of(i * L, L)
        q = q_v[pl.ds(r0, L)]
        i3 = lanes3 + i * (3 * L)
        x = plsc.load_gather(pos_v, [i3])
        y = plsc.load_gather(pos_v, [i3 + 1])
        z = plsc.load_gather(pos_v, [i3 + 2])
        x_v[pl.ds(r0, L)] = q * x
        y_v[pl.ds(r0, L)] = q * y
        z_v[pl.ds(r0, L)] = q * z
        return carry

      lax.fori_loop(0, B // L, step, 0)

      pltpu.sync_copy(q_v, acc0.at[idx_v], add=True)
      pltpu.sync_copy(x_v, acc1.at[idx_v], add=True)
      pltpu.sync_copy(y_v, acc2.at[idx_v], add=True)
      pltpu.sync_copy(z_v, acc3.at[idx_v], add=True)

    plsc.subcore_barrier()

    bufs = (q_v, x_v, y_v, z_v)

    def writeout(row_off, n_rows):
      for c in range(4):
        pltpu.sync_copy(accs[c].at[pl.ds(row_off, n_rows)],
                        bufs[c].at[pl.ds(0, n_rows)])
        dst = pl.multiple_of(cid * 4 * N_SYS + c * N_SYS + row_off, 8)
        pltpu.sync_copy(bufs[c].at[pl.ds(0, n_rows)],
                        out_hbm.at[pl.ds(dst, n_rows)])

    writeout(sid * STRIPE, STRIPE)

    @pl.when(sid == 0)
    def _tail():
      writeout(TAIL_OFF, TAIL)

  return k(charge, pos3, idx, zblock)


def _sc_merge(partials):
  mesh = plsc.VectorSubcoreMesh(core_axis_name="c", subcore_axis_name="s")

  @functools.partial(
      pl.kernel,
      out_type=jax.ShapeDtypeStruct((N_SYS * 4,), jnp.float32),
      mesh=mesh,
      compiler_params=pltpu.CompilerParams(needs_layout_passes=False),
      scratch_types=[
          pltpu.VMEM((RCHUNK,), jnp.float32),
          pltpu.VMEM((RCHUNK,), jnp.float32),
          pltpu.VMEM((RCHUNK * 4,), jnp.float32),
      ],
  )
  def k(p_hbm, out_hbm, a_v, b_v, w_v):
    cid = lax.axis_index("c")
    sid = lax.axis_index("s")
    wid = cid * NS + sid

    lanes4 = lax.iota(jnp.int32, L) * 4

    def merge_rows(row_off, n_rows):
      for c in range(4):
        src_a = pl.multiple_of(c * N_SYS + row_off, 8)
        src_b = pl.multiple_of(4 * N_SYS + c * N_SYS + row_off, 8)
        pltpu.sync_copy(p_hbm.at[pl.ds(src_a, n_rows)],
                        a_v.at[pl.ds(0, n_rows)])
        pltpu.sync_copy(p_hbm.at[pl.ds(src_b, n_rows)],
                        b_v.at[pl.ds(0, n_rows)])

        def add_step(i, carry):
          r0 = pl.multiple_of(i * L, L)
          s = a_v[pl.ds(r0, L)] + b_v[pl.ds(r0, L)]
          plsc.store_scatter(w_v, [lanes4 + (i * (4 * L) + c)], s)
          return carry

        lax.fori_loop(0, n_rows // L, add_step, 0)
      dst = pl.multiple_of(row_off * 4, 8)
      pltpu.sync_copy(w_v.at[pl.ds(0, n_rows * 4)],
                      out_hbm.at[pl.ds(dst, n_rows * 4)])

    merge_rows(wid * RCHUNK, RCHUNK)

    @pl.when(wid == 0)
    def _tail():
      merge_rows(NW * RCHUNK, TAIL)

  return k(partials)


def kernel(per_atom_charge, positions, atomic_subsystem_indices,
           per_system_energy):
  idx = atomic_subsystem_indices.astype(jnp.int32)
  pos3 = positions.reshape(-1)
  zblock = jnp.zeros((STRIPE,), jnp.float32)
  partials = _sc_partials(per_atom_charge, pos3, idx, zblock)
  flat = _sc_merge(partials)
  return flat.reshape(N_SYS, 4)
